# baseline (speedup 1.0000x reference)
import functools
import math

import jax
import jax.numpy as jnp
from jax import lax
from jax.experimental import pallas as pl
from jax.experimental.pallas import tpu as pltpu
from jax.experimental.pallas import tpu_sc as plsc

_T = 32768
_D = 1024
_H = 1024
_K = 256
_TB = 2048
_NBLK = _T // _TB

_NT = 16
_EPT = _T // _NT
_NV = _EPT // 16
_MSB = -2147483648
_LN_DECAY = math.log1p(-1e-7)


def _tc_body(vs_ref, v_ref, wih_ref, whh_ref, x_ref, h_ref, wv_ref,
             bih_ref, bhh_ref, alpha_ref, hnew_ref, sv_ref):
    i = pl.program_id(0)
    blk = vs_ref[...]
    alpha_ref[...] = jax.lax.dot_general(
        v_ref[...], blk, (((1,), (1,)), ((), ())),
        preferred_element_type=jnp.float32).reshape(1, 1, _TB)

    @pl.when(i == _NBLK - 1)
    def _gru():
        gi = jnp.dot(wih_ref[...], x_ref[...],
                     preferred_element_type=jnp.float32) + bih_ref[...]
        gh = jnp.dot(whh_ref[...], h_ref[...],
                     preferred_element_type=jnp.float32) + bhh_ref[...]
        i_r, i_z, i_n = gi[:_H], gi[_H:2 * _H], gi[2 * _H:]
        h_r, h_z, h_n = gh[:_H], gh[_H:2 * _H], gh[2 * _H:]
        r = jax.nn.sigmoid(i_r + h_r)
        z = jax.nn.sigmoid(i_z + h_z)
        n = jnp.tanh(i_n + r * h_n)
        hnew_ref[...] = (1.0 - z) * n + z * h_ref[...]
        sv_ref[...] = jnp.dot(wv_ref[...], v_ref[...].reshape(_D, 1),
                              preferred_element_type=jnp.float32)


def _tc_part(vs, v_row, w_ih, w_hh, x_col, h_col, wv_row, b_ih_col, b_hh_col):
    return pl.pallas_call(
        _tc_body,
        grid=(_NBLK,),
        in_specs=[
            pl.BlockSpec((_TB, _D), lambda i: (i, 0)),
            pl.BlockSpec((1, _D), lambda i: (0, 0)),
            pl.BlockSpec((3 * _H, _D + 1), lambda i: (0, 0)),
            pl.BlockSpec((3 * _H, _H), lambda i: (0, 0)),
            pl.BlockSpec((_D + 1, 1), lambda i: (0, 0)),
            pl.BlockSpec((_H, 1), lambda i: (0, 0)),
            pl.BlockSpec((1, _D), lambda i: (0, 0)),
            pl.BlockSpec((3 * _H, 1), lambda i: (0, 0)),
            pl.BlockSpec((3 * _H, 1), lambda i: (0, 0)),
        ],
        out_specs=[
            pl.BlockSpec((1, 1, _TB), lambda i: (i, 0, 0)),
            pl.BlockSpec((_H, 1), lambda i: (0, 0)),
            pl.BlockSpec((1, 1), lambda i: (0, 0)),
        ],
        out_shape=[
            jax.ShapeDtypeStruct((_NBLK, 1, _TB), jnp.float32),
            jax.ShapeDtypeStruct((_H, 1), jnp.float32),
            jax.ShapeDtypeStruct((1, 1), jnp.float32),
        ],
    )(vs, v_row, w_ih, w_hh, x_col, h_col, wv_row, b_ih_col, b_hh_col)


def _bcast16(x, dtype):
    return lax.broadcast_in_dim(jnp.asarray(x, dtype), (16,), ())


def _bfly(x, op, scratch_ref, lane):
    for d in (1, 2, 4, 8):
        scratch_ref[pl.ds(0, 16)] = x
        y = plsc.load_gather(scratch_ref, [lane ^ d])
        x = op(x, y)
    return x


def _sc_body(alpha_hbm, ts_hbm, taux_hbm, oidx_hbm, ow_hbm, oz_hbm,
             alpha_v, ts_v, key_v, hist_v, allh_v, gtot_v, maxall_v,
             cntall_v, sel_v, eq_v, w_v, t16_v,
             tmp_v, cnt16_v, pos16_v, dat16_v, wdat16_v, red_f, red_i,
             sh_max, sh_hist0, sh_hist1, sh_cnt_gt, sh_cnt_eq,
             dma_sem):
    sid = lax.axis_index("s")
    base = sid * _EPT
    lane = lax.iota(jnp.int32, 16)
    zeros_i = jnp.zeros((16,), jnp.int32)
    ones_i = jnp.ones((16,), jnp.int32)
    c255 = jnp.full((16,), 255, jnp.int32)
    msb16 = jnp.full((16,), _MSB, jnp.int32)

    pltpu.sync_copy(alpha_hbm.at[pl.ds(base, _EPT)], alpha_v)
    pltpu.sync_copy(ts_hbm.at[pl.ds(base, _EPT)], ts_v)
    pltpu.sync_copy(taux_hbm, t16_v)

    def _keys(i, mx):
        a = alpha_v[pl.ds(i * 16, 16)]
        ib = lax.bitcast_convert_type(a, jnp.int32)
        key = jnp.where(ib >= 0, ib, ib ^ jnp.int32(0x7FFFFFFF))
        key_v[pl.ds(i * 16, 16)] = key ^ msb16
        return jnp.maximum(mx, a)
    mx = lax.fori_loop(0, _NV, _keys, jnp.full((16,), -jnp.inf, jnp.float32))
    tmp_v[...] = _bfly(mx, jnp.maximum, red_f, lane)
    pltpu.sync_copy(tmp_v, sh_max.at[pl.ds(sid * 16, 16)])
    plsc.subcore_barrier()
    pltpu.sync_copy(sh_max, maxall_v)

    def _gmax(r, m):
        return jnp.maximum(m, maxall_v[pl.ds(r * 16, 16)])
    m_vec = lax.fori_loop(0, _NT, _gmax,
                          jnp.full((16,), -jnp.inf, jnp.float32))

    prefix = jnp.int32(0)
    mask_hi = jnp.int32(0)
    krem = jnp.int32(_K)
    for rnd in range(4):
        shift = 24 - 8 * rnd
        shift16 = jnp.full((16,), shift, jnp.int32)

        def _clr(j, _):
            hist_v[pl.ds(j * 16, 16)] = zeros_i
            return 0
        lax.fori_loop(0, 256, _clr, 0)

        pfx16 = _bcast16(prefix, jnp.int32)
        mhi16 = _bcast16(mask_hi, jnp.int32)

        def _scan(i, _):
            k2 = key_v[pl.ds(i * 16, 16)]
            cand = (k2 & mhi16) == pfx16
            dig = lax.shift_right_logical(k2, shift16) & c255
            idx = lane * 256 + dig
            plsc.addupdate_scatter(hist_v, [idx], ones_i, mask=cand)
            return 0
        lax.fori_loop(0, _NV, _scan, 0)

        def _mrg(g, _):
            def _acc(l, v):
                return v + hist_v[pl.ds(l * 256 + g * 16, 16)]
            gtot_v[pl.ds(g * 16, 16)] = lax.fori_loop(0, 16, _acc, zeros_i)
            return 0
        lax.fori_loop(0, 16, _mrg, 0)

        sh_h = sh_hist0 if rnd % 2 == 0 else sh_hist1
        pltpu.sync_copy(gtot_v, sh_h.at[pl.ds(sid * 256, 256)])
        plsc.subcore_barrier()
        pltpu.sync_copy(sh_h, allh_v)

        def _gt(g, _):
            def _acc(tt, v):
                return v + allh_v[pl.ds(tt * 256 + g * 16, 16)]
            gtot_v[pl.ds(g * 16, 16)] = lax.fori_loop(0, _NT, _acc, zeros_i)
            return 0
        lax.fori_loop(0, 16, _gt, 0)

        krem16 = _bcast16(krem, jnp.int32)

        def _find(gi, carry):
            above, b_star = carry
            g = 15 - gi
            tv = gtot_v[pl.ds(g * 16, 16)]
            suf = lax.rev(plsc.cumsum(lax.rev(tv, (0,))), (0,))
            cum = suf + above
            m = cum >= krem16
            digs = g * 16 + lane
            bg = _bfly(jnp.where(m, digs, -1), jnp.maximum, red_i, lane)
            return (above + _bfly(tv, jnp.add, red_i, lane),
                    jnp.maximum(b_star, bg))
        _, b_star16 = lax.fori_loop(
            0, 16, _find, (zeros_i, jnp.full((16,), -1, jnp.int32)))

        def _extract(gi, carry):
            above, cum_b, t_b = carry
            g = 15 - gi
            tv = gtot_v[pl.ds(g * 16, 16)]
            suf = lax.rev(plsc.cumsum(lax.rev(tv, (0,))), (0,))
            cum = suf + above
            sel = (g * 16 + lane) == b_star16
            cum_b = cum_b + _bfly(jnp.where(sel, cum, 0), jnp.add,
                                  red_i, lane)
            t_b = t_b + _bfly(jnp.where(sel, tv, 0), jnp.add, red_i, lane)
            return above + _bfly(tv, jnp.add, red_i, lane), cum_b, t_b
        _, cum_b16, t_b16 = lax.fori_loop(
            0, 16, _extract, (zeros_i, zeros_i, zeros_i))

        krem = krem - (cum_b16[0] - t_b16[0])
        prefix = prefix | lax.shift_left(b_star16[0], jnp.int32(shift))
        mask_hi = mask_hi | lax.shift_left(jnp.int32(255), jnp.int32(shift))

    thr = prefix
    thr_s = thr ^ jnp.int32(_MSB)
    thr16 = _bcast16(thr, jnp.int32)
    thr_s16 = _bcast16(thr_s, jnp.int32)

    def _cmp(i, carry):
        off_g, off_e = carry
        k2 = key_v[pl.ds(i * 16, 16)]
        sk = k2 ^ msb16
        gt = sk > thr_s16
        eq = k2 == thr16
        gidx = base + i * 16 + lane
        mi = gt.astype(jnp.int32)
        cs = plsc.cumsum(mi)
        plsc.store_scatter(sel_v, [off_g + cs - mi], gidx, mask=gt)
        mie = eq.astype(jnp.int32)
        cse = plsc.cumsum(mie)
        pose = off_e + cse - mie
        plsc.store_scatter(eq_v, [pose], gidx,
                           mask=eq & (pose < jnp.full((16,), _K, jnp.int32)))
        return (off_g + plsc.all_reduce_population_count(gt)[0],
                off_e + plsc.all_reduce_population_count(eq)[0])
    n_gt, n_eq = lax.fori_loop(0, _NV, _cmp, (jnp.int32(0), jnp.int32(0)))

    cnt16_v[...] = _bcast16(n_gt, jnp.int32)
    pltpu.sync_copy(cnt16_v, sh_cnt_gt.at[pl.ds(sid * 16, 16)])
    cnt16_v[...] = _bcast16(n_eq, jnp.int32)
    pltpu.sync_copy(cnt16_v, sh_cnt_eq.at[pl.ds(sid * 16, 16)])
    plsc.subcore_barrier()

    def _lanes(src):
        def _g(r, acc):
            return jnp.where(lane == r, cntall_v[pl.ds(r * 16, 16)], acc)
        pltpu.sync_copy(src, cntall_v)
        return lax.fori_loop(0, _NT, _g, zeros_i)
    vec_gt = _lanes(sh_cnt_gt)
    vec_eq = _lanes(sh_cnt_eq)
    total_gt = _bfly(vec_gt, jnp.add, red_i, lane)[0]
    e_need = jnp.int32(_K) - total_gt
    eq_before = _bfly(jnp.where(lane < sid, vec_eq, 0), jnp.add,
                      red_i, lane)[0]
    my_take = jnp.clip(e_need - eq_before, 0, n_eq)
    m_t = n_gt + my_take

    def _app(c, _):
        @pl.when(c * 16 < my_take)
        def _():
            valid = (c * 16 + lane) < my_take
            plsc.store_scatter(sel_v, [n_gt + c * 16 + lane],
                               eq_v[pl.ds(c * 16, 16)], mask=valid)
        return 0
    lax.fori_loop(0, 16, _app, 0)

    tvec = t16_v[...]
    lnc = jnp.full((16,), _LN_DECAY, jnp.float32)

    def _wts(c, zacc):
        sl = sel_v[pl.ds(c * 16, 16)]
        valid = (c * 16 + lane) < m_t
        lidx = sl - base
        a = plsc.load_gather(alpha_v, [lidx], mask=valid)
        tsg = plsc.load_gather(ts_v, [lidx], mask=valid)
        dec = jnp.exp(lnc * (tvec - tsg))
        e = jnp.where(valid, jnp.exp(a * dec - m_vec), 0.0)
        w_v[pl.ds(c * 16, 16)] = e
        sel_v[pl.ds(c * 16, 16)] = jnp.where(valid, sl, base)
        return zacc + e
    zacc = lax.fori_loop(0, 16, _wts, jnp.zeros((16,), jnp.float32))
    z_t16 = _bfly(zacc, jnp.add, red_f, lane)

    excl = plsc.cumsum(vec_eq) - vec_eq
    take_vec = jnp.clip(_bcast16(e_need, jnp.int32) - excl, 0, vec_eq)
    m_vec_all = vec_gt + take_vec
    my_off = _bfly(jnp.where(lane < sid, m_vec_all, 0), jnp.add,
                   red_i, lane)[0]

    def _out(c, _):
        @pl.when(c * 16 < m_t)
        def _():
            valid = (c * 16 + lane) < m_t
            pos16_v[...] = jnp.where(valid, my_off + c * 16 + lane,
                                     _K + lane)
            dat16_v[...] = sel_v[pl.ds(c * 16, 16)]
            wdat16_v[...] = w_v[pl.ds(c * 16, 16)]
            pltpu.async_copy(dat16_v, oidx_hbm.at[pos16_v], dma_sem).wait()
            pltpu.async_copy(wdat16_v, ow_hbm.at[pos16_v], dma_sem).wait()
        return 0
    lax.fori_loop(0, 16, _out, 0)

    tmp_v[...] = z_t16
    pltpu.sync_copy(tmp_v, sh_max.at[pl.ds(sid * 16, 16)])
    plsc.subcore_barrier()

    @pl.when(sid == 0)
    def _fin():
        pltpu.sync_copy(sh_max, maxall_v)

        def _sum(r, v):
            return v + maxall_v[pl.ds(r * 16, 16)]
        z_all = lax.fori_loop(0, _NT, _sum, jnp.zeros((16,), jnp.float32))
        tmp_v[...] = z_all
        pltpu.sync_copy(tmp_v, oz_hbm)


@functools.partial(
    pl.kernel,
    out_type=(
        jax.ShapeDtypeStruct((_K + 16,), jnp.int32),
        jax.ShapeDtypeStruct((_K + 16,), jnp.float32),
        jax.ShapeDtypeStruct((16,), jnp.float32),
    ),
    mesh=plsc.VectorSubcoreMesh(core_axis_name="c", subcore_axis_name="s",
                                num_cores=1),
    compiler_params=pltpu.CompilerParams(needs_layout_passes=False),
    scratch_types=[
        pltpu.VMEM((_EPT,), jnp.float32),
        pltpu.VMEM((_EPT,), jnp.float32),
        pltpu.VMEM((_EPT,), jnp.int32),
        pltpu.VMEM((4096,), jnp.int32),
        pltpu.VMEM((4096,), jnp.int32),
        pltpu.VMEM((256,), jnp.int32),
        pltpu.VMEM((256,), jnp.float32),
        pltpu.VMEM((256,), jnp.int32),
        pltpu.VMEM((256,), jnp.int32),
        pltpu.VMEM((256,), jnp.int32),
        pltpu.VMEM((256,), jnp.float32),
        pltpu.VMEM((16,), jnp.float32),
        pltpu.VMEM((16,), jnp.float32),
        pltpu.VMEM((16,), jnp.int32),
        pltpu.VMEM((16,), jnp.int32),
        pltpu.VMEM((16,), jnp.int32),
        pltpu.VMEM((16,), jnp.float32),
        pltpu.VMEM((128,), jnp.float32),
        pltpu.VMEM((128,), jnp.int32),
        pltpu.VMEM_SHARED((256,), jnp.float32),
        pltpu.VMEM_SHARED((4096,), jnp.int32),
        pltpu.VMEM_SHARED((4096,), jnp.int32),
        pltpu.VMEM_SHARED((256,), jnp.int32),
        pltpu.VMEM_SHARED((256,), jnp.int32),
        pltpu.SemaphoreType.DMA,
    ],
)
def _sc_part(alpha_hbm, ts_hbm, taux_hbm, oidx_hbm, ow_hbm, oz_hbm, *rest):
    _sc_body(alpha_hbm, ts_hbm, taux_hbm, oidx_hbm, ow_hbm, oz_hbm, *rest)


_RING = 16


def _gather_body(idx_smem, w_smem, z_smem, sv_smem, b_smem, hs_ref, wh_ref,
                 out_ref, buf, sems):
    def _start(k, slot):
        pltpu.make_async_copy(
            hs_ref.at[pl.ds(idx_smem[k], 1), :],
            buf.at[pl.ds(slot, 1), :],
            sems.at[slot]).start()

    for slot in range(_RING):
        _start(slot, slot)

    def _body(k0, acc):
        for slot in range(_RING):
            k = k0 * _RING + slot
            pltpu.make_async_copy(
                hs_ref.at[pl.ds(idx_smem[k], 1), :],
                buf.at[pl.ds(slot, 1), :],
                sems.at[slot]).wait()
            acc = acc + w_smem[k] * buf[pl.ds(slot, 1), :]

            @pl.when(k0 < (_K // _RING) - 1)
            def _():
                _start(k + _RING, slot)
        return acc
    acc = lax.fori_loop(0, _K // _RING, _body,
                        jnp.zeros((1, _H), jnp.float32))
    s = jnp.sum(acc * wh_ref[...])
    out_ref[...] = (sv_smem[0, 0] + s / z_smem[0]
                    + b_smem[0]).reshape(1, 1)


def _tc_gather(sel_idx, sel_w, z16, sv, b_score_arr, hs2, wh_row):
    return pl.pallas_call(
        _gather_body,
        in_specs=[
            pl.BlockSpec(memory_space=pltpu.SMEM),
            pl.BlockSpec(memory_space=pltpu.SMEM),
            pl.BlockSpec(memory_space=pltpu.SMEM),
            pl.BlockSpec(memory_space=pltpu.SMEM),
            pl.BlockSpec(memory_space=pltpu.SMEM),
            pl.BlockSpec(memory_space=pl.ANY),
            pl.BlockSpec((1, _D)),
        ],
        out_specs=pl.BlockSpec((1, 1)),
        out_shape=jax.ShapeDtypeStruct((1, 1), jnp.float32),
        scratch_shapes=[
            pltpu.VMEM((_RING, _H), jnp.float32),
            pltpu.SemaphoreType.DMA((_RING,)),
        ],
    )(sel_idx, sel_w, z16, sv, b_score_arr, hs2, wh_row)


def kernel(v, s, t, vs, hs, ts, W_score, b_score, W_ih, W_hh, b_ih, b_hh):
    v_row = v.reshape(1, _D)
    x_col = jnp.concatenate([v, s]).reshape(_D + 1, 1)
    h_col = hs[-1, 0].reshape(_H, 1)
    wv_row = W_score[:, :_D]

    alpha_blk, hnew_col, sv = _tc_part(
        vs, v_row, W_ih, W_hh, x_col, h_col, wv_row,
        b_ih.reshape(3 * _H, 1), b_hh.reshape(3 * _H, 1))
    alpha = alpha_blk.reshape(_T)

    taux = jnp.broadcast_to(t, (16,))
    hs2 = hs.reshape(_T, _H)
    sel_idx, sel_w, z16 = _sc_part(alpha, ts, taux)

    score = _tc_gather(sel_idx, sel_w, z16, sv, b_score, hs2,
                       W_score[:, _D:])
    h_new = hnew_col.reshape(1, 1, _H)
    return (score, h_new)


# speedup vs baseline: 2.1611x; 2.1611x over previous
import functools
import math

import jax
import jax.numpy as jnp
from jax import lax
from jax.experimental import pallas as pl
from jax.experimental.pallas import tpu as pltpu
from jax.experimental.pallas import tpu_sc as plsc

_T = 32768
_D = 1024
_H = 1024
_K = 256
_TB = 2048
_NBLK = _T // _TB

_NT = 16
_EPT = _T // _NT
_NV = _EPT // 16
_MSB = -2147483648
_LN_DECAY = math.log1p(-1e-7)


def _alpha_body(vs_ref, v_ref, alpha_ref):
    blk = vs_ref[...]
    alpha_ref[...] = jax.lax.dot_general(
        v_ref[...], blk, (((1,), (1,)), ((), ())),
        preferred_element_type=jnp.float32).reshape(1, 1, _TB)


def _alpha_part(vs, v_row):
    return pl.pallas_call(
        _alpha_body,
        grid=(_NBLK,),
        in_specs=[
            pl.BlockSpec((_TB, _D), lambda i: (i, 0)),
            pl.BlockSpec((1, _D), lambda i: (0, 0)),
        ],
        out_specs=pl.BlockSpec((1, 1, _TB), lambda i: (i, 0, 0)),
        out_shape=jax.ShapeDtypeStruct((_NBLK, 1, _TB), jnp.float32),
    )(vs, v_row)


def _gru_body(v_ref, wih_ref, whh_ref, x_ref, h_ref, wv_ref,
              bih_ref, bhh_ref, hnew_ref, sv_ref):
    gi = jnp.dot(wih_ref[...], x_ref[...],
                 preferred_element_type=jnp.float32) + bih_ref[...]
    gh = jnp.dot(whh_ref[...], h_ref[...],
                 preferred_element_type=jnp.float32) + bhh_ref[...]
    i_r, i_z, i_n = gi[:_H], gi[_H:2 * _H], gi[2 * _H:]
    h_r, h_z, h_n = gh[:_H], gh[_H:2 * _H], gh[2 * _H:]
    r = jax.nn.sigmoid(i_r + h_r)
    z = jax.nn.sigmoid(i_z + h_z)
    n = jnp.tanh(i_n + r * h_n)
    hnew_ref[...] = (1.0 - z) * n + z * h_ref[...]
    sv_ref[...] = jnp.dot(wv_ref[...], v_ref[...].reshape(_D, 1),
                          preferred_element_type=jnp.float32)


def _gru_part(v_row, w_ih, w_hh, x_col, h_col, wv_row, b_ih_col, b_hh_col):
    return pl.pallas_call(
        _gru_body,
        out_shape=[
            jax.ShapeDtypeStruct((_H, 1), jnp.float32),
            jax.ShapeDtypeStruct((1, 1), jnp.float32),
        ],
    )(v_row, w_ih, w_hh, x_col, h_col, wv_row, b_ih_col, b_hh_col)


def _bcast16(x, dtype):
    return lax.broadcast_in_dim(jnp.asarray(x, dtype), (16,), ())


def _bfly(x, op, scratch_ref, lane):
    for d in (1, 2, 4, 8):
        scratch_ref[pl.ds(0, 16)] = x
        y = plsc.load_gather(scratch_ref, [lane ^ d])
        x = op(x, y)
    return x


def _sc_body(alpha_hbm, ts_hbm, taux_hbm, oidx_hbm, ow_hbm, oz_hbm,
             alpha_v, ts_v, key_v, hist_v, gtot_v, maxall_v,
             cntall_v, sel_v, eq_v, w_v, t16_v,
             tmp_v, cnt16_v, pos16_v, dat16_v, wdat16_v, idx255_v,
             red_f, red_i,
             sh_max, sh_g, sh_cnt_gt, sh_cnt_eq,
             dma_sem):
    sid = lax.axis_index("s")
    base = sid * _EPT
    lane = lax.iota(jnp.int32, 16)
    zeros_i = jnp.zeros((16,), jnp.int32)
    ones_i = jnp.ones((16,), jnp.int32)
    c255 = jnp.full((16,), 255, jnp.int32)
    msb16 = jnp.full((16,), _MSB, jnp.int32)

    pltpu.sync_copy(alpha_hbm.at[pl.ds(base, _EPT)], alpha_v)
    pltpu.sync_copy(ts_hbm.at[pl.ds(base, _EPT)], ts_v)
    pltpu.sync_copy(taux_hbm, t16_v)

    def _iot(c, _):
        idx255_v[pl.ds(c * 16, 16)] = c * 16 + lane
        cntall_v[pl.ds(c * 16, 16)] = zeros_i
        return 0
    lax.fori_loop(0, 16, _iot, 0)

    @pl.when(sid == 0)
    def _zero_shared():
        for rr in range(4):
            pltpu.sync_copy(cntall_v, sh_g.at[pl.ds(rr * 256, 256)])

    def _keys(i, mx):
        for k in range(4):
            a = alpha_v[pl.ds((i * 4 + k) * 16, 16)]
            ib = lax.bitcast_convert_type(a, jnp.int32)
            key = jnp.where(ib >= 0, ib, ib ^ jnp.int32(0x7FFFFFFF))
            key_v[pl.ds((i * 4 + k) * 16, 16)] = key ^ msb16
            mx = jnp.maximum(mx, a)
        return mx
    mx = lax.fori_loop(0, _NV // 4, _keys,
                       jnp.full((16,), -jnp.inf, jnp.float32))
    tmp_v[...] = _bfly(mx, jnp.maximum, red_f, lane)
    pltpu.sync_copy(tmp_v, sh_max.at[pl.ds(sid * 16, 16)])
    plsc.subcore_barrier()
    pltpu.sync_copy(sh_max, maxall_v)

    def _gmax(r, m):
        return jnp.maximum(m, maxall_v[pl.ds(r * 16, 16)])
    m_vec = lax.fori_loop(0, _NT, _gmax,
                          jnp.full((16,), -jnp.inf, jnp.float32))

    prefix = jnp.int32(0)
    mask_hi = jnp.int32(0)
    krem = jnp.int32(_K)
    for rnd in range(4):
        shift = 24 - 8 * rnd
        shift16 = jnp.full((16,), shift, jnp.int32)

        def _clr(j, _):
            for k in range(8):
                hist_v[pl.ds((j * 8 + k) * 16, 16)] = zeros_i
            return 0
        lax.fori_loop(0, 32, _clr, 0)

        pfx16 = _bcast16(prefix, jnp.int32)
        mhi16 = _bcast16(mask_hi, jnp.int32)

        def _scan(i, _):
            for k in range(4):
                k2 = key_v[pl.ds((i * 4 + k) * 16, 16)]
                cand = (k2 & mhi16) == pfx16
                dig = lax.shift_right_logical(k2, shift16) & c255
                idx = lane * 256 + dig
                plsc.addupdate_scatter(hist_v, [idx], ones_i, mask=cand)
            return 0
        lax.fori_loop(0, _NV // 4, _scan, 0)

        def _mrg(g, _):
            v = zeros_i
            for l in range(16):
                v = v + hist_v[pl.ds(l * 256 + g * 16, 16)]
            gtot_v[pl.ds(g * 16, 16)] = v
            return 0
        lax.fori_loop(0, 16, _mrg, 0)

        def _iotr(c, _):
            idx255_v[pl.ds(c * 16, 16)] = rnd * 256 + c * 16 + lane
            return 0
        lax.fori_loop(0, 16, _iotr, 0)
        pltpu.sync_copy(gtot_v, sh_g.at[idx255_v], add=True)
        plsc.subcore_barrier()
        pltpu.sync_copy(sh_g.at[pl.ds(rnd * 256, 256)], gtot_v)

        krem16 = _bcast16(krem, jnp.int32)

        def _gsum(j, v):
            return v + plsc.load_gather(gtot_v, [lane * 16 + j])
        gs = lax.fori_loop(0, 16, _gsum, zeros_i)
        sufg = lax.rev(plsc.cumsum(lax.rev(gs, (0,))), (0,))
        m1 = (sufg >= krem16).astype(jnp.int32)
        sstar = 15 - plsc.all_reduce_ffs(lax.rev(m1, (0,)) > 0)

        def _pick(vec, pos16):
            red_i[pl.ds(0, 16)] = vec
            return plsc.load_gather(red_i, [pos16])
        above = jnp.where(sstar >= 15, 0,
                          _pick(sufg, jnp.minimum(sstar + 1, 15)))
        tvs = plsc.load_gather(gtot_v, [sstar * 16 + lane])
        suf_in = lax.rev(plsc.cumsum(lax.rev(tvs, (0,))), (0,)) + above
        m2 = (suf_in >= krem16).astype(jnp.int32)
        b_in = 15 - plsc.all_reduce_ffs(lax.rev(m2, (0,)) > 0)
        b_star16 = sstar * 16 + b_in
        cum_b16 = _pick(suf_in, b_in)
        t_b16 = _pick(tvs, b_in)

        krem = krem - (cum_b16[0] - t_b16[0])
        prefix = prefix | lax.shift_left(b_star16[0], jnp.int32(shift))
        mask_hi = mask_hi | lax.shift_left(jnp.int32(255), jnp.int32(shift))

    thr = prefix
    thr_s = thr ^ jnp.int32(_MSB)
    thr16 = _bcast16(thr, jnp.int32)
    thr_s16 = _bcast16(thr_s, jnp.int32)

    def _cmp(i, carry):
        off_g, off_e = carry
        k2 = key_v[pl.ds(i * 16, 16)]
        sk = k2 ^ msb16
        gt = sk > thr_s16
        eq = k2 == thr16
        gidx = base + i * 16 + lane
        mi = gt.astype(jnp.int32)
        cs = plsc.cumsum(mi)
        plsc.store_scatter(sel_v, [off_g + cs - mi], gidx, mask=gt)
        mie = eq.astype(jnp.int32)
        cse = plsc.cumsum(mie)
        pose = off_e + cse - mie
        plsc.store_scatter(eq_v, [pose], gidx,
                           mask=eq & (pose < jnp.full((16,), _K, jnp.int32)))
        return (off_g + plsc.all_reduce_population_count(gt)[0],
                off_e + plsc.all_reduce_population_count(eq)[0])
    n_gt, n_eq = lax.fori_loop(0, _NV, _cmp, (jnp.int32(0), jnp.int32(0)))

    cnt16_v[...] = _bcast16(n_gt, jnp.int32)
    pltpu.sync_copy(cnt16_v, sh_cnt_gt.at[pl.ds(sid * 16, 16)])
    cnt16_v[...] = _bcast16(n_eq, jnp.int32)
    pltpu.sync_copy(cnt16_v, sh_cnt_eq.at[pl.ds(sid * 16, 16)])
    plsc.subcore_barrier()

    def _lanes(src):
        def _g(r, acc):
            return jnp.where(lane == r, cntall_v[pl.ds(r * 16, 16)], acc)
        pltpu.sync_copy(src, cntall_v)
        return lax.fori_loop(0, _NT, _g, zeros_i)
    vec_gt = _lanes(sh_cnt_gt)
    vec_eq = _lanes(sh_cnt_eq)
    total_gt = _bfly(vec_gt, jnp.add, red_i, lane)[0]
    e_need = jnp.int32(_K) - total_gt
    eq_before = _bfly(jnp.where(lane < sid, vec_eq, 0), jnp.add,
                      red_i, lane)[0]
    my_take = jnp.clip(e_need - eq_before, 0, n_eq)
    m_t = n_gt + my_take

    def _app(c, _):
        @pl.when(c * 16 < my_take)
        def _():
            valid = (c * 16 + lane) < my_take
            plsc.store_scatter(sel_v, [n_gt + c * 16 + lane],
                               eq_v[pl.ds(c * 16, 16)], mask=valid)
        return 0
    lax.fori_loop(0, 16, _app, 0)

    tvec = t16_v[...]
    lnc = jnp.full((16,), _LN_DECAY, jnp.float32)

    def _wts(c, zacc):
        sl = sel_v[pl.ds(c * 16, 16)]
        valid = (c * 16 + lane) < m_t
        lidx = sl - base
        a = plsc.load_gather(alpha_v, [lidx], mask=valid)
        tsg = plsc.load_gather(ts_v, [lidx], mask=valid)
        dec = jnp.exp(lnc * (tvec - tsg))
        e = jnp.where(valid, jnp.exp(a * dec - m_vec), 0.0)
        w_v[pl.ds(c * 16, 16)] = e
        sel_v[pl.ds(c * 16, 16)] = jnp.where(valid, sl, base)
        return zacc + e
    zacc = lax.fori_loop(0, 16, _wts, jnp.zeros((16,), jnp.float32))
    z_t16 = _bfly(zacc, jnp.add, red_f, lane)

    excl = plsc.cumsum(vec_eq) - vec_eq
    take_vec = jnp.clip(_bcast16(e_need, jnp.int32) - excl, 0, vec_eq)
    m_vec_all = vec_gt + take_vec
    my_off = _bfly(jnp.where(lane < sid, m_vec_all, 0), jnp.add,
                   red_i, lane)[0]

    def _out(c, _):
        @pl.when(c * 16 < m_t)
        def _():
            valid = (c * 16 + lane) < m_t
            pos16_v[...] = jnp.where(valid, my_off + c * 16 + lane,
                                     _K + lane)
            dat16_v[...] = sel_v[pl.ds(c * 16, 16)]
            wdat16_v[...] = w_v[pl.ds(c * 16, 16)]
            pltpu.async_copy(dat16_v, oidx_hbm.at[pos16_v], dma_sem).wait()
            pltpu.async_copy(wdat16_v, ow_hbm.at[pos16_v], dma_sem).wait()
        return 0
    lax.fori_loop(0, 16, _out, 0)

    tmp_v[...] = z_t16
    pltpu.sync_copy(tmp_v, sh_max.at[pl.ds(sid * 16, 16)])
    plsc.subcore_barrier()

    @pl.when(sid == 0)
    def _fin():
        pltpu.sync_copy(sh_max, maxall_v)

        def _sum(r, v):
            return v + maxall_v[pl.ds(r * 16, 16)]
        z_all = lax.fori_loop(0, _NT, _sum, jnp.zeros((16,), jnp.float32))
        tmp_v[...] = z_all
        pltpu.sync_copy(tmp_v, oz_hbm)


@functools.partial(
    pl.kernel,
    out_type=(
        jax.ShapeDtypeStruct((_K + 16,), jnp.int32),
        jax.ShapeDtypeStruct((_K + 16,), jnp.float32),
        jax.ShapeDtypeStruct((16,), jnp.float32),
    ),
    mesh=plsc.VectorSubcoreMesh(core_axis_name="c", subcore_axis_name="s",
                                num_cores=1),
    compiler_params=pltpu.CompilerParams(needs_layout_passes=False),
    scratch_types=[
        pltpu.VMEM((_EPT,), jnp.float32),
        pltpu.VMEM((_EPT,), jnp.float32),
        pltpu.VMEM((_EPT,), jnp.int32),
        pltpu.VMEM((4096,), jnp.int32),
        pltpu.VMEM((256,), jnp.int32),
        pltpu.VMEM((256,), jnp.float32),
        pltpu.VMEM((256,), jnp.int32),
        pltpu.VMEM((256,), jnp.int32),
        pltpu.VMEM((256,), jnp.int32),
        pltpu.VMEM((256,), jnp.float32),
        pltpu.VMEM((16,), jnp.float32),
        pltpu.VMEM((16,), jnp.float32),
        pltpu.VMEM((16,), jnp.int32),
        pltpu.VMEM((16,), jnp.int32),
        pltpu.VMEM((16,), jnp.int32),
        pltpu.VMEM((16,), jnp.float32),
        pltpu.VMEM((256,), jnp.int32),
        pltpu.VMEM((128,), jnp.float32),
        pltpu.VMEM((128,), jnp.int32),
        pltpu.VMEM_SHARED((256,), jnp.float32),
        pltpu.VMEM_SHARED((1024,), jnp.int32),
        pltpu.VMEM_SHARED((256,), jnp.int32),
        pltpu.VMEM_SHARED((256,), jnp.int32),
        pltpu.SemaphoreType.DMA,
    ],
)
def _sc_part(alpha_hbm, ts_hbm, taux_hbm, oidx_hbm, ow_hbm, oz_hbm, *rest):
    _sc_body(alpha_hbm, ts_hbm, taux_hbm, oidx_hbm, ow_hbm, oz_hbm, *rest)


_RING = 16


def _gather_body(idx_smem, w_smem, z_smem, sv_smem, b_smem, hs_ref, wh_ref,
                 out_ref, buf, sems):
    def _start(k, slot):
        pltpu.make_async_copy(
            hs_ref.at[pl.ds(idx_smem[k], 1), :, :],
            buf.at[pl.ds(slot, 1), :, :],
            sems.at[slot]).start()

    for slot in range(_RING):
        _start(slot, slot)

    def _body(k0, acc):
        for slot in range(_RING):
            k = k0 * _RING + slot
            pltpu.make_async_copy(
                hs_ref.at[pl.ds(idx_smem[k], 1), :, :],
                buf.at[pl.ds(slot, 1), :, :],
                sems.at[slot]).wait()
            acc = acc + w_smem[k] * buf[pl.ds(slot, 1), :, :].reshape(1, _H)

            @pl.when(k0 < (_K // _RING) - 1)
            def _():
                _start(k + _RING, slot)
        return acc
    acc = lax.fori_loop(0, _K // _RING, _body,
                        jnp.zeros((1, _H), jnp.float32))
    s = jnp.sum(acc * wh_ref[...])
    out_ref[...] = (sv_smem[0, 0] + s / z_smem[0]
                    + b_smem[0]).reshape(1, 1)


def _tc_gather(sel_idx, sel_w, z16, sv, b_score_arr, hs2, wh_row):
    return pl.pallas_call(
        _gather_body,
        in_specs=[
            pl.BlockSpec(memory_space=pltpu.SMEM),
            pl.BlockSpec(memory_space=pltpu.SMEM),
            pl.BlockSpec(memory_space=pltpu.SMEM),
            pl.BlockSpec(memory_space=pltpu.SMEM),
            pl.BlockSpec(memory_space=pltpu.SMEM),
            pl.BlockSpec(memory_space=pl.ANY),
            pl.BlockSpec((1, _D)),
        ],
        out_specs=pl.BlockSpec((1, 1)),
        out_shape=jax.ShapeDtypeStruct((1, 1), jnp.float32),
        scratch_shapes=[
            pltpu.VMEM((_RING, 1, _H), jnp.float32),
            pltpu.SemaphoreType.DMA((_RING,)),
        ],
    )(sel_idx, sel_w, z16, sv, b_score_arr, hs2, wh_row)


def kernel(v, s, t, vs, hs, ts, W_score, b_score, W_ih, W_hh, b_ih, b_hh):
    v_row = v.reshape(1, _D)
    x_col = jnp.concatenate([v, s]).reshape(_D + 1, 1)
    h_col = hs[-1, 0].reshape(_H, 1)
    wv_row = W_score[:, :_D]

    alpha_blk = _alpha_part(vs, v_row)
    hnew_col, sv = _gru_part(
        v_row, W_ih, W_hh, x_col, h_col, wv_row,
        b_ih.reshape(3 * _H, 1), b_hh.reshape(3 * _H, 1))
    alpha = alpha_blk.reshape(_T)

    taux = jnp.broadcast_to(t, (16,))
    sel_idx, sel_w, z16 = _sc_part(alpha, ts, taux)

    score = _tc_gather(sel_idx, sel_w, z16, sv, b_score, hs,
                       W_score[:, _D:])
    h_new = hnew_col.reshape(1, 1, _H)
    return (score, h_new)
